# baseline (speedup 1.0000x reference)
import functools

import jax
import jax.numpy as jnp
from jax import lax
from jax.experimental import pallas as pl
from jax.experimental.pallas import tpu as pltpu
from jax.experimental.pallas import tpu_sc as plsc

N_NODES = 10000
D = 128
NC, NS = 2, 16
NW = NC * NS
B = 128
CPP = 40
NPASS = 2
ACC_ROWS = 10240
ROWS_PER_TILE = ACC_ROWS // NS


def _sc_partials(x, src3, dst3, rsrc, rdst, zeros):
    nfull, rem = src3.shape[1], rsrc.shape[0]
    mesh = plsc.VectorSubcoreMesh(core_axis_name="c", subcore_axis_name="s")

    @functools.partial(
        pl.kernel,
        out_type=jax.ShapeDtypeStruct((NC, ACC_ROWS, D), jnp.float32),
        mesh=mesh,
        scratch_types=[
            pltpu.VMEM((CPP, B), jnp.int32),
            pltpu.VMEM((CPP, B), jnp.int32),
            pltpu.VMEM((B, D), jnp.float32),
            pltpu.VMEM((B, D), jnp.float32),
            pltpu.VMEM_SHARED((ACC_ROWS, D), jnp.float32),
            pltpu.SemaphoreType.DMA,
            pltpu.SemaphoreType.DMA,
            pltpu.SemaphoreType.DMA,
            pltpu.SemaphoreType.DMA,
        ],
    )
    def k(x_hbm, src_hbm, dst_hbm, rsrc_hbm, rdst_hbm, zeros_hbm, out_hbm,
          src_v, dst_v, rows0, rows1, acc, sg0, sg1, ss0, ss1):
        c = lax.axis_index("c")
        s = lax.axis_index("s")
        w = s * NC + c

        with jax.named_scope("zinit"):
            pltpu.sync_copy(
                zeros_hbm, acc.at[pl.ds(s * ROWS_PER_TILE, ROWS_PER_TILE)])
            plsc.subcore_barrier()

        def wg(buf, sem, j):
            pltpu.make_async_copy(x_hbm.at[src_v.at[j]], buf, sem).wait()

        def ws(buf, sem, j):
            pltpu.make_async_copy(buf, acc.at[dst_v.at[j]], sem).wait()

        def run_pass(row0, cpp):
            pltpu.sync_copy(src_hbm.at[w, pl.ds(row0, cpp)],
                            src_v.at[pl.ds(0, cpp)])
            pltpu.sync_copy(dst_hbm.at[w, pl.ds(row0, cpp)],
                            dst_v.at[pl.ds(0, cpp)])

            pltpu.async_copy(x_hbm.at[src_v.at[0]], rows0, sg0)
            wg(rows0, sg0, 0)
            pltpu.async_copy(rows0, acc.at[dst_v.at[0]], ss0, add=True)
            pltpu.async_copy(x_hbm.at[src_v.at[1]], rows1, sg1)

            def body(i, cr):
                j1 = 2 * i + 1
                j2 = j1 + 1
                wg(rows1, sg1, j1)
                pltpu.async_copy(rows1, acc.at[dst_v.at[j1]], ss1, add=True)
                ws(rows0, ss0, j1 - 1)
                pltpu.async_copy(x_hbm.at[src_v.at[j2]], rows0, sg0)
                wg(rows0, sg0, j2)
                pltpu.async_copy(rows0, acc.at[dst_v.at[j2]], ss0, add=True)
                ws(rows1, ss1, j1)
                pltpu.async_copy(x_hbm.at[src_v.at[j2 + 1]], rows1, sg1)
                return cr

            lax.fori_loop(0, cpp // 2 - 1, body, 0)
            jt = cpp - 1
            wg(rows1, sg1, jt)
            pltpu.async_copy(rows1, acc.at[dst_v.at[jt]], ss1, add=True)
            ws(rows0, ss0, jt - 1)
            ws(rows1, ss1, jt)

        with jax.named_scope("mainloop"):
            done = 0
            while done < nfull:
                cpp = min(CPP, nfull - done)
                run_pass(done, cpp)
                done += cpp

            @pl.when(w < rem)
            def _():
                pltpu.sync_copy(rsrc_hbm.at[w], src_v.at[0])
                pltpu.sync_copy(rdst_hbm.at[w], dst_v.at[0])
                pltpu.async_copy(x_hbm.at[src_v.at[0]], rows0, sg0).wait()
                pltpu.sync_copy(rows0, acc.at[dst_v.at[0]], add=True)

            plsc.subcore_barrier()

        with jax.named_scope("export"):
            pltpu.sync_copy(
                acc.at[pl.ds(s * ROWS_PER_TILE, ROWS_PER_TILE)],
                out_hbm.at[c, pl.ds(s * ROWS_PER_TILE, ROWS_PER_TILE)],
            )

    return k(x, src3, dst3, rsrc, rdst, zeros)


def _combine(partials):
    blk = 1000

    def body(p_ref, o_ref):
        o_ref[...] = p_ref[0] + p_ref[1]

    return pl.pallas_call(
        body,
        grid=(N_NODES // blk,),
        in_specs=[pl.BlockSpec((NC, blk, D), lambda i: (0, i, 0))],
        out_specs=pl.BlockSpec((blk, D), lambda i: (i, 0)),
        out_shape=jax.ShapeDtypeStruct((N_NODES, D), jnp.float32),
    )(partials)


def kernel(x, edge_index):
    src = edge_index[0].astype(jnp.int32)
    dst = edge_index[1].astype(jnp.int32)
    e = src.shape[0]
    assert e % B == 0, e
    nch = e // B
    nfull = nch // NW
    cut = nfull * NW * B
    rem = nch - nfull * NW
    src3 = src[:cut].reshape(NW, nfull, B)
    dst3 = dst[:cut].reshape(NW, nfull, B)
    rsrc = src[cut:].reshape(rem, B)
    rdst = dst[cut:].reshape(rem, B)
    zeros = jnp.zeros((ROWS_PER_TILE, D), jnp.float32)
    partials = _sc_partials(x, src3, dst3, rsrc, rdst, zeros)
    return _combine(partials)


# speedup vs baseline: 1.0892x; 1.0892x over previous
import functools

import jax
import jax.numpy as jnp
from jax import lax
from jax.experimental import pallas as pl
from jax.experimental.pallas import tpu as pltpu
from jax.experimental.pallas import tpu_sc as plsc

N_NODES = 10000
D = 128
NC, NS = 2, 16
NW = NC * NS
B = 128
CPP = 40
NPASS = 2
ACC_ROWS = 10240
ROWS_PER_TILE = ACC_ROWS // NS


def _sc_partials(x, src3, dst3, zeros):
    npass, cpp = src3.shape[1], src3.shape[2]
    mesh = plsc.VectorSubcoreMesh(core_axis_name="c", subcore_axis_name="s")

    @functools.partial(
        pl.kernel,
        out_type=jax.ShapeDtypeStruct((NC, ACC_ROWS, D), jnp.float32),
        mesh=mesh,
        scratch_types=[
            pltpu.VMEM((cpp, B), jnp.int32),
            pltpu.VMEM((cpp, B), jnp.int32),
            pltpu.VMEM((B, D), jnp.float32),
            pltpu.VMEM((B, D), jnp.float32),
            pltpu.VMEM_SHARED((ACC_ROWS, D), jnp.float32),
            pltpu.SemaphoreType.DMA,
            pltpu.SemaphoreType.DMA,
            pltpu.SemaphoreType.DMA,
            pltpu.SemaphoreType.DMA,
        ],
    )
    def k(x_hbm, src_hbm, dst_hbm, zeros_hbm, out_hbm, src_v, dst_v, rows0,
          rows1, acc, sg0, sg1, ss0, ss1):
        c = lax.axis_index("c")
        s = lax.axis_index("s")
        w = s * NC + c

        with jax.named_scope("zinit"):
            pltpu.sync_copy(
                zeros_hbm, acc.at[pl.ds(s * ROWS_PER_TILE, ROWS_PER_TILE)])
            plsc.subcore_barrier()

        def wg(buf, sem, j):
            pltpu.make_async_copy(x_hbm.at[src_v.at[j]], buf, sem).wait()

        def ws(buf, sem, j):
            pltpu.make_async_copy(buf, acc.at[dst_v.at[j]], sem).wait()

        with jax.named_scope("mainloop"):
            def do_pass(p, carry):
                pltpu.sync_copy(src_hbm.at[w, p], src_v)
                pltpu.sync_copy(dst_hbm.at[w, p], dst_v)

                pltpu.async_copy(x_hbm.at[src_v.at[0]], rows0, sg0)
                wg(rows0, sg0, 0)
                pltpu.async_copy(rows0, acc.at[dst_v.at[0]], ss0, add=True)
                pltpu.async_copy(x_hbm.at[src_v.at[1]], rows1, sg1)

                def body(i, cr):
                    j1 = 2 * i + 1
                    j2 = j1 + 1
                    wg(rows1, sg1, j1)
                    pltpu.async_copy(rows1, acc.at[dst_v.at[j1]], ss1, add=True)
                    ws(rows0, ss0, j1 - 1)
                    pltpu.async_copy(x_hbm.at[src_v.at[j2]], rows0, sg0)
                    wg(rows0, sg0, j2)
                    pltpu.async_copy(rows0, acc.at[dst_v.at[j2]], ss0, add=True)
                    ws(rows1, ss1, j1)
                    pltpu.async_copy(x_hbm.at[src_v.at[j2 + 1]], rows1, sg1)
                    return cr

                lax.fori_loop(0, cpp // 2 - 1, body, 0)
                jt = cpp - 1
                wg(rows1, sg1, jt)
                pltpu.async_copy(rows1, acc.at[dst_v.at[jt]], ss1, add=True)
                ws(rows0, ss0, jt - 1)
                ws(rows1, ss1, jt)
                return carry

            lax.fori_loop(0, npass, do_pass, 0)
            plsc.subcore_barrier()

        with jax.named_scope("export"):
            pltpu.sync_copy(
                acc.at[pl.ds(s * ROWS_PER_TILE, ROWS_PER_TILE)],
                out_hbm.at[c, pl.ds(s * ROWS_PER_TILE, ROWS_PER_TILE)],
            )

    return k(x, src3, dst3, zeros)


def _combine(partials):
    blk = 1000

    def body(p_ref, o_ref):
        o_ref[...] = p_ref[0] + p_ref[1]

    return pl.pallas_call(
        body,
        grid=(N_NODES // blk,),
        in_specs=[pl.BlockSpec((NC, blk, D), lambda i: (0, i, 0))],
        out_specs=pl.BlockSpec((blk, D), lambda i: (i, 0)),
        out_shape=jax.ShapeDtypeStruct((N_NODES, D), jnp.float32),
    )(partials)


def kernel(x, edge_index):
    src = edge_index[0].astype(jnp.int32)
    dst = edge_index[1].astype(jnp.int32)
    e = src.shape[0]
    g = NW * NPASS * CPP * B
    assert e <= g, (e, g)
    pad = g - e
    if pad:
        it = jnp.arange(pad, dtype=jnp.int32)
        src = jnp.concatenate([src, it % N_NODES])
        dst = jnp.concatenate([dst, N_NODES + it % (ACC_ROWS - N_NODES)])
    src3 = src.reshape(NW, NPASS, CPP, B)
    dst3 = dst.reshape(NW, NPASS, CPP, B)
    zeros = jnp.zeros((ROWS_PER_TILE, D), jnp.float32)
    partials = _sc_partials(x, src3, dst3, zeros)
    return _combine(partials)
